# baseline (speedup 1.0000x reference)
import functools

import jax
import jax.numpy as jnp
from jax import lax
from jax.experimental import pallas as pl
from jax.experimental.pallas import tpu as pltpu
from jax.experimental.pallas import tpu_sc as plsc

NC = 2
NS = 16
L = 16
CHUNK = 128
DW = 16

_sc_mesh = functools.partial(
    plsc.VectorSubcoreMesh, core_axis_name="c", subcore_axis_name="s"
)
_sc_params = pltpu.CompilerParams(use_tc_tiling_on_sc=False)


def _fill_vmem(ref, rows, width, value):
  @pl.loop(0, rows)
  def _(r):
    @pl.loop(0, width, step=L)
    def _(j):
      ref[r, pl.ds(j, L)] = jnp.full((L,), value, jnp.float32)


def _row_split(n):
  rps = (n // NS) // 8 * 8
  return rps, n - NS * rps


def _zero_range(acc, src, base, rows):
  nfull = rows // CHUNK
  tail = rows - nfull * CHUNK

  @pl.loop(0, nfull)
  def _(k):
    pltpu.sync_copy(src, acc.at[pl.ds(base + k * CHUNK, CHUNK)])

  if tail:
    pltpu.sync_copy(src.at[pl.ds(0, tail)], acc.at[pl.ds(base + nfull * CHUNK, tail)])


def _zero_shared(acc, src, sid, n):
  rps, rem = _row_split(n)
  _zero_range(acc, src, sid * rps, rps)
  if rem:
    @pl.when(sid == 0)
    def _():
      _zero_range(acc, src, NS * rps, rem)


def _writeout(acc, out_hbm, cid, sid, n):
  rps, rem = _row_split(n)
  sl = pl.ds(sid * rps, rps)
  pltpu.sync_copy(acc.at[sl], out_hbm.at[cid, sl])
  if rem:
    @pl.when(sid == 0)
    def _():
      sl2 = pl.ds(NS * rps, rem)
      pltpu.sync_copy(acc.at[sl2], out_hbm.at[cid, sl2])


def _make_deg(n, e, nbuf=6):
  n_chunks = e // CHUNK
  cpc = n_chunks // NC
  kb = cpc // NS
  ng = kb // nbuf
  rem = kb - ng * nbuf
  tail_tiles = cpc - kb * NS

  def body(edge_hbm, out_hbm, acc, idx_v, ones_v, zero_v, sem_i, sem_s):
    cid = lax.axis_index("c")
    sid = lax.axis_index("s")
    _fill_vmem(ones_v, CHUNK, DW, 1.0)
    _fill_vmem(zero_v, CHUNK, DW, 0.0)
    _zero_shared(acc, zero_v, sid, n)
    plsc.subcore_barrier()

    def base_of(k):
      return (cid * cpc + sid + k * NS) * CHUNK

    def issue_idx(k, b):
      pltpu.async_copy(edge_hbm.at[0, pl.ds(base_of(k), CHUNK)], idx_v.at[b],
                       sem_i.at[b])

    def wait_idx(k, b):
      pltpu.make_async_copy(edge_hbm.at[0, pl.ds(base_of(k), CHUNK)],
                            idx_v.at[b], sem_i.at[b]).wait()

    def issue_scatter(b):
      pltpu.async_copy(ones_v, acc.at[idx_v.at[b]], sem_s.at[b], add=True)

    def wait_scatter(b):
      pltpu.make_async_copy(ones_v, acc.at[idx_v.at[b]], sem_s.at[b]).wait()

    for b in range(nbuf):
      issue_idx(b, b)

    @pl.loop(0, ng)
    def _(g):
      @pl.when(g > 0)
      def _():
        for b in range(nbuf):
          wait_scatter(b)
          issue_idx(g * nbuf + b, b)
      for b in range(nbuf):
        wait_idx(g * nbuf + b, b)
        issue_scatter(b)

    for b in range(nbuf):
      wait_scatter(b)
    for j in range(rem):
      k = ng * nbuf + j
      issue_idx(k, 0)
      wait_idx(k, 0)
      issue_scatter(0)
      wait_scatter(0)
    if tail_tiles:
      @pl.when(sid < tail_tiles)
      def _():
        base = (cid * cpc + kb * NS + sid) * CHUNK
        pltpu.sync_copy(edge_hbm.at[0, pl.ds(base, CHUNK)], idx_v.at[0])
        pltpu.sync_copy(ones_v, acc.at[idx_v.at[0]], add=True)

    plsc.subcore_barrier()
    rps, rem_rows = _row_split(n)
    sl = pl.ds(sid * rps, rps)
    eight = pl.ds(0, 8)
    pltpu.sync_copy(acc.at[sl, eight], out_hbm.at[cid, sl])
    if rem_rows:
      @pl.when(sid == 0)
      def _():
        sl2 = pl.ds(NS * rps, rem_rows)
        pltpu.sync_copy(acc.at[sl2, eight], out_hbm.at[cid, sl2])

  return pl.kernel(
      body,
      out_type=jax.ShapeDtypeStruct((NC, n, 8), jnp.float32),
      mesh=_sc_mesh(),
      scratch_types=[
          pltpu.VMEM_SHARED((n, DW), jnp.float32),
          pltpu.VMEM((nbuf, CHUNK), jnp.int32),
          pltpu.VMEM((CHUNK, DW), jnp.float32),
          pltpu.VMEM((CHUNK, DW), jnp.float32),
          pltpu.SemaphoreType.DMA((nbuf,)),
          pltpu.SemaphoreType.DMA((nbuf,)),
      ],
      compiler_params=_sc_params,
  )


def _make_agg(n, e, f, nbuf, tc_tiling, stage_y=False):
  n_chunks = e // CHUNK
  cpc = n_chunks // NC
  kb = cpc // NS
  ng = kb // nbuf
  rem = kb - ng * nbuf
  tail_tiles = cpc - kb * NS

  def body(y_hbm, edge_hbm, out_hbm, acc, row_v, col_v, msg_v,
           sem_r, sem_c, sem_g, sem_s, *maybe_ysh):
    cid = lax.axis_index("c")
    sid = lax.axis_index("s")
    _fill_vmem(msg_v.at[0], CHUNK, f, 0.0)
    _zero_shared(acc, msg_v.at[0], sid, n)
    if stage_y:
      y_sh = maybe_ysh[0]
      rps, rem_rows = _row_split(n)
      sl = pl.ds(sid * rps, rps)
      pltpu.sync_copy(y_hbm.at[sl], y_sh.at[sl])
      if rem_rows:
        @pl.when(sid == 0)
        def _():
          sl2 = pl.ds(NS * rps, rem_rows)
          pltpu.sync_copy(y_hbm.at[sl2], y_sh.at[sl2])
      y_src = y_sh
    else:
      y_src = y_hbm
    plsc.subcore_barrier()

    def base_of(k):
      return (cid * cpc + sid + k * NS) * CHUNK

    def issue_idx(k, b):
      pltpu.async_copy(edge_hbm.at[0, pl.ds(base_of(k), CHUNK)], row_v.at[b],
                       sem_r.at[b])
      pltpu.async_copy(edge_hbm.at[1, pl.ds(base_of(k), CHUNK)], col_v.at[b],
                       sem_c.at[b])

    def wait_idx(k, b):
      pltpu.make_async_copy(edge_hbm.at[0, pl.ds(base_of(k), CHUNK)],
                            row_v.at[b], sem_r.at[b]).wait()
      pltpu.make_async_copy(edge_hbm.at[1, pl.ds(base_of(k), CHUNK)],
                            col_v.at[b], sem_c.at[b]).wait()

    def issue_gather(b):
      pltpu.async_copy(y_src.at[row_v.at[b]], msg_v.at[b], sem_g.at[b])

    def wait_gather(b):
      pltpu.make_async_copy(y_src.at[row_v.at[b]], msg_v.at[b],
                            sem_g.at[b]).wait()

    def issue_scatter(b):
      pltpu.async_copy(msg_v.at[b], acc.at[col_v.at[b]], sem_s.at[b], add=True)

    def wait_scatter(b):
      pltpu.make_async_copy(msg_v.at[b], acc.at[col_v.at[b]],
                            sem_s.at[b]).wait()

    for b in range(nbuf):
      issue_idx(b, b)

    @pl.loop(0, ng)
    def _(g):
      @pl.when(g > 0)
      def _():
        for b in range(nbuf):
          wait_scatter(b)
          issue_idx(g * nbuf + b, b)
      for b in range(nbuf):
        wait_idx(g * nbuf + b, b)
        issue_gather(b)
      for b in range(nbuf):
        wait_gather(b)
        issue_scatter(b)

    for b in range(nbuf):
      wait_scatter(b)
    for j in range(rem):
      k = ng * nbuf + j
      issue_idx(k, 0)
      wait_idx(k, 0)
      issue_gather(0)
      wait_gather(0)
      issue_scatter(0)
      wait_scatter(0)
    if tail_tiles:
      @pl.when(sid < tail_tiles)
      def _():
        base = (cid * cpc + kb * NS + sid) * CHUNK
        pltpu.sync_copy(edge_hbm.at[0, pl.ds(base, CHUNK)], row_v.at[0])
        pltpu.sync_copy(edge_hbm.at[1, pl.ds(base, CHUNK)], col_v.at[0])
        pltpu.async_copy(y_src.at[row_v.at[0]], msg_v.at[0], sem_g.at[0]).wait()
        pltpu.sync_copy(msg_v.at[0], acc.at[col_v.at[0]], add=True)

    plsc.subcore_barrier()
    _writeout(acc, out_hbm, cid, sid, n)

  return pl.kernel(
      body,
      out_type=jax.ShapeDtypeStruct((NC, n, f), jnp.float32),
      mesh=_sc_mesh(),
      scratch_types=[
          pltpu.VMEM_SHARED((n, f), jnp.float32),
          pltpu.VMEM((nbuf, CHUNK), jnp.int32),
          pltpu.VMEM((nbuf, CHUNK), jnp.int32),
          pltpu.VMEM((nbuf, CHUNK, f), jnp.float32),
          pltpu.SemaphoreType.DMA((nbuf,)),
          pltpu.SemaphoreType.DMA((nbuf,)),
          pltpu.SemaphoreType.DMA((nbuf,)),
          pltpu.SemaphoreType.DMA((nbuf,)),
      ] + ([pltpu.VMEM_SHARED((n, f), jnp.float32)] if stage_y else []),
      compiler_params=pltpu.CompilerParams(use_tc_tiling_on_sc=tc_tiling),
  )


_CONTRACT_T = (((1,), (1,)), ((), ()))


def _mm_body(x_ref, w_ref, b_ref, o_ref):
  xw = lax.dot_general(x_ref[...], w_ref[...], _CONTRACT_T,
                       preferred_element_type=jnp.float32)
  o_ref[...] = xw + b_ref[...]


def _scale_body(degp_ref, xw_ref, dinv_ref, y_ref):
  deg = jnp.sum(degp_ref[0] + degp_ref[1], axis=-1, keepdims=True) * 0.125 + 1.0
  dinv = lax.rsqrt(deg)
  dinv_ref[...] = dinv
  y_ref[...] = dinv * xw_ref[...]


def _layer2_body(acc_ref, y1_ref, dinv_ref, w_ref, b_ref, y2_ref):
  dinv = dinv_ref[...]
  h = jnp.maximum(dinv * (acc_ref[0] + acc_ref[1] + y1_ref[...]), 0.0)
  hw = lax.dot_general(h, w_ref[...], _CONTRACT_T,
                       preferred_element_type=jnp.float32)
  y2_ref[...] = dinv * (hw + b_ref[...])


def _final_body(acc_ref, y2_ref, dinv_ref, o_ref):
  z = dinv_ref[...] * (acc_ref[0] + acc_ref[1] + y2_ref[...])
  m = jnp.max(z, axis=1, keepdims=True)
  ez = jnp.exp(z - m)
  lse = jnp.log(jnp.sum(ez, axis=1, keepdims=True)) + m
  o_ref[...] = z - lse


def kernel(x, edge_index, W1, b1, W2, b2):
  n, f_in = x.shape
  e = edge_index.shape[1]
  f_hid = W1.shape[0]
  f_out = W2.shape[0]
  b1r = b1.reshape(1, f_hid)
  b2r = b2.reshape(1, f_out)

  degp = _make_deg(n, e)(edge_index)
  xw1 = pl.pallas_call(
      _mm_body,
      out_shape=jax.ShapeDtypeStruct((n, f_hid), jnp.float32),
  )(x, W1, b1r)

  dinv, y1 = pl.pallas_call(
      _scale_body,
      out_shape=[jax.ShapeDtypeStruct((n, 1), jnp.float32),
                 jax.ShapeDtypeStruct((n, f_hid), jnp.float32)],
  )(degp, xw1)

  acc1 = _make_agg(n, e, f_hid, 3, tc_tiling=True)(y1, edge_index)

  y2 = pl.pallas_call(
      _layer2_body,
      out_shape=jax.ShapeDtypeStruct((n, f_out), jnp.float32),
  )(acc1, y1, dinv, W2, b2r)

  acc2 = _make_agg(n, e, f_out, 6, tc_tiling=False)(y2, edge_index)

  out = pl.pallas_call(
      _final_body,
      out_shape=jax.ShapeDtypeStruct((n, f_out), jnp.float32),
  )(acc2, y2, dinv)

  return out


# speedup vs baseline: 1.0034x; 1.0034x over previous
import functools

import jax
import jax.numpy as jnp
from jax import lax
from jax.experimental import pallas as pl
from jax.experimental.pallas import tpu as pltpu
from jax.experimental.pallas import tpu_sc as plsc

NC = 2
NS = 16
L = 16
CHUNK = 128
DW = 16

_sc_mesh = functools.partial(
    plsc.VectorSubcoreMesh, core_axis_name="c", subcore_axis_name="s"
)
_sc_params = pltpu.CompilerParams(use_tc_tiling_on_sc=False)


def _fill_vmem(ref, rows, width, value):
  @pl.loop(0, rows)
  def _(r):
    @pl.loop(0, width, step=L)
    def _(j):
      ref[r, pl.ds(j, L)] = jnp.full((L,), value, jnp.float32)


def _row_split(n):
  rps = (n // NS) // 8 * 8
  return rps, n - NS * rps


def _zero_range(acc, src, base, rows):
  nfull = rows // CHUNK
  tail = rows - nfull * CHUNK

  @pl.loop(0, nfull)
  def _(k):
    pltpu.sync_copy(src, acc.at[pl.ds(base + k * CHUNK, CHUNK)])

  if tail:
    pltpu.sync_copy(src.at[pl.ds(0, tail)], acc.at[pl.ds(base + nfull * CHUNK, tail)])


def _zero_shared(acc, src, sid, n):
  rps, rem = _row_split(n)
  _zero_range(acc, src, sid * rps, rps)
  if rem:
    @pl.when(sid == 0)
    def _():
      _zero_range(acc, src, NS * rps, rem)


def _writeout(acc, out_hbm, cid, sid, n):
  rps, rem = _row_split(n)
  sl = pl.ds(sid * rps, rps)
  pltpu.sync_copy(acc.at[sl], out_hbm.at[cid, sl])
  if rem:
    @pl.when(sid == 0)
    def _():
      sl2 = pl.ds(NS * rps, rem)
      pltpu.sync_copy(acc.at[sl2], out_hbm.at[cid, sl2])


def _make_deg(n, e, nbuf=6):
  n_chunks = e // CHUNK
  cpc = n_chunks // NC
  kb = cpc // NS
  ng = kb // nbuf
  rem = kb - ng * nbuf
  tail_tiles = cpc - kb * NS

  def body(edge_hbm, out_hbm, acc, idx_v, ones_v, zero_v, sem_i, sem_s):
    cid = lax.axis_index("c")
    sid = lax.axis_index("s")
    _fill_vmem(ones_v, CHUNK, DW, 1.0)
    _fill_vmem(zero_v, CHUNK, DW, 0.0)
    _zero_shared(acc, zero_v, sid, n)
    plsc.subcore_barrier()

    def base_of(k):
      return (cid * cpc + sid + k * NS) * CHUNK

    def issue_idx(k, b):
      pltpu.async_copy(edge_hbm.at[0, pl.ds(base_of(k), CHUNK)], idx_v.at[b],
                       sem_i.at[b])

    def wait_idx(k, b):
      pltpu.make_async_copy(edge_hbm.at[0, pl.ds(base_of(k), CHUNK)],
                            idx_v.at[b], sem_i.at[b]).wait()

    def issue_scatter(b):
      pltpu.async_copy(ones_v, acc.at[idx_v.at[b]], sem_s.at[b], add=True)

    def wait_scatter(b):
      pltpu.make_async_copy(ones_v, acc.at[idx_v.at[b]], sem_s.at[b]).wait()

    for b in range(nbuf):
      issue_idx(b, b)

    @pl.loop(0, ng)
    def _(g):
      @pl.when(g > 0)
      def _():
        for b in range(nbuf):
          wait_scatter(b)
          issue_idx(g * nbuf + b, b)
      for b in range(nbuf):
        wait_idx(g * nbuf + b, b)
        issue_scatter(b)

    for b in range(nbuf):
      wait_scatter(b)
    for j in range(rem):
      k = ng * nbuf + j
      issue_idx(k, 0)
      wait_idx(k, 0)
      issue_scatter(0)
      wait_scatter(0)
    if tail_tiles:
      @pl.when(sid < tail_tiles)
      def _():
        base = (cid * cpc + kb * NS + sid) * CHUNK
        pltpu.sync_copy(edge_hbm.at[0, pl.ds(base, CHUNK)], idx_v.at[0])
        pltpu.sync_copy(ones_v, acc.at[idx_v.at[0]], add=True)

    plsc.subcore_barrier()
    rps, rem_rows = _row_split(n)
    sl = pl.ds(sid * rps, rps)
    eight = pl.ds(0, 8)
    pltpu.sync_copy(acc.at[sl, eight], out_hbm.at[cid, sl])
    if rem_rows:
      @pl.when(sid == 0)
      def _():
        sl2 = pl.ds(NS * rps, rem_rows)
        pltpu.sync_copy(acc.at[sl2, eight], out_hbm.at[cid, sl2])

  return pl.kernel(
      body,
      out_type=jax.ShapeDtypeStruct((NC, n, 8), jnp.float32),
      mesh=_sc_mesh(),
      scratch_types=[
          pltpu.VMEM_SHARED((n, DW), jnp.float32),
          pltpu.VMEM((nbuf, CHUNK), jnp.int32),
          pltpu.VMEM((CHUNK, DW), jnp.float32),
          pltpu.VMEM((CHUNK, DW), jnp.float32),
          pltpu.SemaphoreType.DMA((nbuf,)),
          pltpu.SemaphoreType.DMA((nbuf,)),
      ],
      compiler_params=_sc_params,
  )


def _make_agg(n, e, f, nbuf, tc_tiling, stage_y=False):
  n_chunks = e // CHUNK
  cpc = n_chunks // NC
  kb = cpc // NS
  ng = kb // nbuf
  rem = kb - ng * nbuf
  tail_tiles = cpc - kb * NS

  def body(y_hbm, edge_hbm, out_hbm, acc, row_v, col_v, msg_v,
           sem_r, sem_c, sem_g, sem_s, *maybe_ysh):
    cid = lax.axis_index("c")
    sid = lax.axis_index("s")
    _fill_vmem(msg_v.at[0], CHUNK, f, 0.0)
    _zero_shared(acc, msg_v.at[0], sid, n)
    if stage_y:
      y_sh = maybe_ysh[0]
      rps, rem_rows = _row_split(n)
      sl = pl.ds(sid * rps, rps)
      pltpu.sync_copy(y_hbm.at[sl], y_sh.at[sl])
      if rem_rows:
        @pl.when(sid == 0)
        def _():
          sl2 = pl.ds(NS * rps, rem_rows)
          pltpu.sync_copy(y_hbm.at[sl2], y_sh.at[sl2])
      y_src = y_sh
    else:
      y_src = y_hbm
    plsc.subcore_barrier()

    def base_of(k):
      return (cid * cpc + sid + k * NS) * CHUNK

    def issue_idx(k, b):
      pltpu.async_copy(edge_hbm.at[0, pl.ds(base_of(k), CHUNK)], row_v.at[b],
                       sem_r.at[b])
      pltpu.async_copy(edge_hbm.at[1, pl.ds(base_of(k), CHUNK)], col_v.at[b],
                       sem_c.at[b])

    def wait_idx(k, b):
      pltpu.make_async_copy(edge_hbm.at[0, pl.ds(base_of(k), CHUNK)],
                            row_v.at[b], sem_r.at[b]).wait()
      pltpu.make_async_copy(edge_hbm.at[1, pl.ds(base_of(k), CHUNK)],
                            col_v.at[b], sem_c.at[b]).wait()

    def issue_gather(b):
      pltpu.async_copy(y_src.at[row_v.at[b]], msg_v.at[b], sem_g.at[b])

    def wait_gather(b):
      pltpu.make_async_copy(y_src.at[row_v.at[b]], msg_v.at[b],
                            sem_g.at[b]).wait()

    def issue_scatter(b):
      pltpu.async_copy(msg_v.at[b], acc.at[col_v.at[b]], sem_s.at[b], add=True)

    def wait_scatter(b):
      pltpu.make_async_copy(msg_v.at[b], acc.at[col_v.at[b]],
                            sem_s.at[b]).wait()

    for b in range(nbuf):
      issue_idx(b, b)

    @pl.loop(0, ng)
    def _(g):
      @pl.when(g > 0)
      def _():
        for b in range(nbuf):
          wait_scatter(b)
          issue_idx(g * nbuf + b, b)
      for b in range(nbuf):
        wait_idx(g * nbuf + b, b)
        issue_gather(b)
      for b in range(nbuf):
        wait_gather(b)
        issue_scatter(b)

    for b in range(nbuf):
      wait_scatter(b)
    for j in range(rem):
      k = ng * nbuf + j
      issue_idx(k, 0)
      wait_idx(k, 0)
      issue_gather(0)
      wait_gather(0)
      issue_scatter(0)
      wait_scatter(0)
    if tail_tiles:
      @pl.when(sid < tail_tiles)
      def _():
        base = (cid * cpc + kb * NS + sid) * CHUNK
        pltpu.sync_copy(edge_hbm.at[0, pl.ds(base, CHUNK)], row_v.at[0])
        pltpu.sync_copy(edge_hbm.at[1, pl.ds(base, CHUNK)], col_v.at[0])
        pltpu.async_copy(y_src.at[row_v.at[0]], msg_v.at[0], sem_g.at[0]).wait()
        pltpu.sync_copy(msg_v.at[0], acc.at[col_v.at[0]], add=True)

    plsc.subcore_barrier()
    _writeout(acc, out_hbm, cid, sid, n)

  return pl.kernel(
      body,
      out_type=jax.ShapeDtypeStruct((NC, n, f), jnp.float32),
      mesh=_sc_mesh(),
      scratch_types=[
          pltpu.VMEM_SHARED((n, f), jnp.float32),
          pltpu.VMEM((nbuf, CHUNK), jnp.int32),
          pltpu.VMEM((nbuf, CHUNK), jnp.int32),
          pltpu.VMEM((nbuf, CHUNK, f), jnp.float32),
          pltpu.SemaphoreType.DMA((nbuf,)),
          pltpu.SemaphoreType.DMA((nbuf,)),
          pltpu.SemaphoreType.DMA((nbuf,)),
          pltpu.SemaphoreType.DMA((nbuf,)),
      ] + ([pltpu.VMEM_SHARED((n, f), jnp.float32)] if stage_y else []),
      compiler_params=pltpu.CompilerParams(use_tc_tiling_on_sc=tc_tiling),
  )


_CONTRACT_T = (((1,), (1,)), ((), ()))


def _mm_scale_body(x_ref, w_ref, b_ref, degp_ref, dinv_ref, y_ref):
  deg = jnp.sum(degp_ref[0] + degp_ref[1], axis=-1, keepdims=True) * 0.125 + 1.0
  dinv = lax.rsqrt(deg)
  dinv_ref[...] = dinv
  xw = lax.dot_general(x_ref[...], w_ref[...], _CONTRACT_T,
                       preferred_element_type=jnp.float32)
  y_ref[...] = dinv * (xw + b_ref[...])


def _layer2_body(acc_ref, y1_ref, dinv_ref, w_ref, b_ref, y2_ref):
  dinv = dinv_ref[...]
  h = jnp.maximum(dinv * (acc_ref[0] + acc_ref[1] + y1_ref[...]), 0.0)
  hw = lax.dot_general(h, w_ref[...], _CONTRACT_T,
                       preferred_element_type=jnp.float32)
  y2_ref[...] = dinv * (hw + b_ref[...])


def _final_body(acc_ref, y2_ref, dinv_ref, o_ref):
  z = dinv_ref[...] * (acc_ref[0] + acc_ref[1] + y2_ref[...])
  m = jnp.max(z, axis=1, keepdims=True)
  ez = jnp.exp(z - m)
  lse = jnp.log(jnp.sum(ez, axis=1, keepdims=True)) + m
  o_ref[...] = z - lse


def kernel(x, edge_index, W1, b1, W2, b2):
  n, f_in = x.shape
  e = edge_index.shape[1]
  f_hid = W1.shape[0]
  f_out = W2.shape[0]
  b1r = b1.reshape(1, f_hid)
  b2r = b2.reshape(1, f_out)

  degp = _make_deg(n, e)(edge_index)

  dinv, y1 = pl.pallas_call(
      _mm_scale_body,
      out_shape=[jax.ShapeDtypeStruct((n, 1), jnp.float32),
                 jax.ShapeDtypeStruct((n, f_hid), jnp.float32)],
  )(x, W1, b1r, degp)

  acc1 = _make_agg(n, e, f_hid, 3, tc_tiling=True)(y1, edge_index)

  y2 = pl.pallas_call(
      _layer2_body,
      out_shape=jax.ShapeDtypeStruct((n, f_out), jnp.float32),
  )(acc1, y1, dinv, W2, b2r)

  acc2 = _make_agg(n, e, f_out, 6, tc_tiling=False)(y2, edge_index)

  out = pl.pallas_call(
      _final_body,
      out_shape=jax.ShapeDtypeStruct((n, f_out), jnp.float32),
  )(acc2, y2, dinv)

  return out


# speedup vs baseline: 1.0240x; 1.0204x over previous
import functools

import jax
import jax.numpy as jnp
from jax import lax
from jax.experimental import pallas as pl
from jax.experimental.pallas import tpu as pltpu
from jax.experimental.pallas import tpu_sc as plsc

NC = 2
NS = 16
L = 16
CHUNK = 128
DW = 16

_sc_mesh = functools.partial(
    plsc.VectorSubcoreMesh, core_axis_name="c", subcore_axis_name="s"
)
_sc_params = pltpu.CompilerParams(use_tc_tiling_on_sc=False)


def _fill_vmem(ref, rows, width, value):
  @pl.loop(0, rows)
  def _(r):
    @pl.loop(0, width, step=L)
    def _(j):
      ref[r, pl.ds(j, L)] = jnp.full((L,), value, jnp.float32)


def _row_split(n):
  rps = (n // NS) // 8 * 8
  return rps, n - NS * rps


def _zero_range(acc, src, base, rows):
  nfull = rows // CHUNK
  tail = rows - nfull * CHUNK

  @pl.loop(0, nfull)
  def _(k):
    pltpu.sync_copy(src, acc.at[pl.ds(base + k * CHUNK, CHUNK)])

  if tail:
    pltpu.sync_copy(src.at[pl.ds(0, tail)], acc.at[pl.ds(base + nfull * CHUNK, tail)])


def _zero_shared(acc, src, sid, n):
  rps, rem = _row_split(n)
  _zero_range(acc, src, sid * rps, rps)
  if rem:
    @pl.when(sid == 0)
    def _():
      _zero_range(acc, src, NS * rps, rem)


def _writeout(acc, out_hbm, cid, sid, n):
  rps, rem = _row_split(n)
  sl = pl.ds(sid * rps, rps)
  pltpu.sync_copy(acc.at[sl], out_hbm.at[cid, sl])
  if rem:
    @pl.when(sid == 0)
    def _():
      sl2 = pl.ds(NS * rps, rem)
      pltpu.sync_copy(acc.at[sl2], out_hbm.at[cid, sl2])


def _make_deg(n, e, nbuf=6):
  n_chunks = e // CHUNK
  cpc = n_chunks // NC
  kb = cpc // NS
  ng = kb // nbuf
  rem = kb - ng * nbuf
  tail_tiles = cpc - kb * NS

  def body(edge_hbm, out_hbm, acc, idx_v, ones_v, zero_v, sem_i, sem_s):
    cid = lax.axis_index("c")
    sid = lax.axis_index("s")
    _fill_vmem(ones_v, CHUNK, DW, 1.0)
    _fill_vmem(zero_v, CHUNK, DW, 0.0)
    _zero_shared(acc, zero_v, sid, n)
    plsc.subcore_barrier()

    def base_of(k):
      return (cid * cpc + sid + k * NS) * CHUNK

    def issue_idx(k, b):
      pltpu.async_copy(edge_hbm.at[0, pl.ds(base_of(k), CHUNK)], idx_v.at[b],
                       sem_i.at[b])

    def wait_idx(k, b):
      pltpu.make_async_copy(edge_hbm.at[0, pl.ds(base_of(k), CHUNK)],
                            idx_v.at[b], sem_i.at[b]).wait()

    def issue_scatter(b):
      pltpu.async_copy(ones_v, acc.at[idx_v.at[b]], sem_s.at[b], add=True)

    def wait_scatter(b):
      pltpu.make_async_copy(ones_v, acc.at[idx_v.at[b]], sem_s.at[b]).wait()

    for b in range(nbuf):
      issue_idx(b, b)

    @pl.loop(0, ng)
    def _(g):
      @pl.when(g > 0)
      def _():
        for b in range(nbuf):
          wait_scatter(b)
          issue_idx(g * nbuf + b, b)
      for b in range(nbuf):
        wait_idx(g * nbuf + b, b)
        issue_scatter(b)

    for b in range(nbuf):
      wait_scatter(b)
    for j in range(rem):
      k = ng * nbuf + j
      issue_idx(k, 0)
      wait_idx(k, 0)
      issue_scatter(0)
      wait_scatter(0)
    if tail_tiles:
      @pl.when(sid < tail_tiles)
      def _():
        base = (cid * cpc + kb * NS + sid) * CHUNK
        pltpu.sync_copy(edge_hbm.at[0, pl.ds(base, CHUNK)], idx_v.at[0])
        pltpu.sync_copy(ones_v, acc.at[idx_v.at[0]], add=True)

    plsc.subcore_barrier()
    _writeout(acc, out_hbm, cid, sid, n)

  return pl.kernel(
      body,
      out_type=jax.ShapeDtypeStruct((NC, n, DW), jnp.float32),
      mesh=_sc_mesh(),
      scratch_types=[
          pltpu.VMEM_SHARED((n, DW), jnp.float32),
          pltpu.VMEM((nbuf, CHUNK), jnp.int32),
          pltpu.VMEM((CHUNK, DW), jnp.float32),
          pltpu.VMEM((CHUNK, DW), jnp.float32),
          pltpu.SemaphoreType.DMA((nbuf,)),
          pltpu.SemaphoreType.DMA((nbuf,)),
      ],
      compiler_params=_sc_params,
  )


def _make_agg(n, e, f, nbuf, tc_tiling, stage_y=False):
  n_chunks = e // CHUNK
  cpc = n_chunks // NC
  kb = cpc // NS
  ng = kb // nbuf
  rem = kb - ng * nbuf
  tail_tiles = cpc - kb * NS

  def body(y_hbm, edge_hbm, out_hbm, acc, row_v, col_v, msg_v,
           sem_r, sem_c, sem_g, sem_s, *maybe_ysh):
    cid = lax.axis_index("c")
    sid = lax.axis_index("s")
    _fill_vmem(msg_v.at[0], CHUNK, f, 0.0)
    _zero_shared(acc, msg_v.at[0], sid, n)
    if stage_y:
      y_sh = maybe_ysh[0]
      rps, rem_rows = _row_split(n)
      sl = pl.ds(sid * rps, rps)
      pltpu.sync_copy(y_hbm.at[sl], y_sh.at[sl])
      if rem_rows:
        @pl.when(sid == 0)
        def _():
          sl2 = pl.ds(NS * rps, rem_rows)
          pltpu.sync_copy(y_hbm.at[sl2], y_sh.at[sl2])
      y_src = y_sh
    else:
      y_src = y_hbm
    plsc.subcore_barrier()

    def base_of(k):
      return (cid * cpc + sid + k * NS) * CHUNK

    def issue_idx(k, b):
      pltpu.async_copy(edge_hbm.at[0, pl.ds(base_of(k), CHUNK)], row_v.at[b],
                       sem_r.at[b])
      pltpu.async_copy(edge_hbm.at[1, pl.ds(base_of(k), CHUNK)], col_v.at[b],
                       sem_c.at[b])

    def wait_idx(k, b):
      pltpu.make_async_copy(edge_hbm.at[0, pl.ds(base_of(k), CHUNK)],
                            row_v.at[b], sem_r.at[b]).wait()
      pltpu.make_async_copy(edge_hbm.at[1, pl.ds(base_of(k), CHUNK)],
                            col_v.at[b], sem_c.at[b]).wait()

    def issue_gather(b):
      pltpu.async_copy(y_src.at[row_v.at[b]], msg_v.at[b], sem_g.at[b])

    def wait_gather(b):
      pltpu.make_async_copy(y_src.at[row_v.at[b]], msg_v.at[b],
                            sem_g.at[b]).wait()

    def issue_scatter(b):
      pltpu.async_copy(msg_v.at[b], acc.at[col_v.at[b]], sem_s.at[b], add=True)

    def wait_scatter(b):
      pltpu.make_async_copy(msg_v.at[b], acc.at[col_v.at[b]],
                            sem_s.at[b]).wait()

    for b in range(nbuf):
      issue_idx(b, b)

    @pl.loop(0, ng)
    def _(g):
      @pl.when(g > 0)
      def _():
        for b in range(nbuf):
          wait_scatter(b)
          issue_idx(g * nbuf + b, b)
      for b in range(nbuf):
        wait_idx(g * nbuf + b, b)
        issue_gather(b)
      for b in range(nbuf):
        wait_gather(b)
        issue_scatter(b)

    for b in range(nbuf):
      wait_scatter(b)
    for j in range(rem):
      k = ng * nbuf + j
      issue_idx(k, 0)
      wait_idx(k, 0)
      issue_gather(0)
      wait_gather(0)
      issue_scatter(0)
      wait_scatter(0)
    if tail_tiles:
      @pl.when(sid < tail_tiles)
      def _():
        base = (cid * cpc + kb * NS + sid) * CHUNK
        pltpu.sync_copy(edge_hbm.at[0, pl.ds(base, CHUNK)], row_v.at[0])
        pltpu.sync_copy(edge_hbm.at[1, pl.ds(base, CHUNK)], col_v.at[0])
        pltpu.async_copy(y_src.at[row_v.at[0]], msg_v.at[0], sem_g.at[0]).wait()
        pltpu.sync_copy(msg_v.at[0], acc.at[col_v.at[0]], add=True)

    plsc.subcore_barrier()
    _writeout(acc, out_hbm, cid, sid, n)

  return pl.kernel(
      body,
      out_type=jax.ShapeDtypeStruct((NC, n, f), jnp.float32),
      mesh=_sc_mesh(),
      scratch_types=[
          pltpu.VMEM_SHARED((n, f), jnp.float32),
          pltpu.VMEM((nbuf, CHUNK), jnp.int32),
          pltpu.VMEM((nbuf, CHUNK), jnp.int32),
          pltpu.VMEM((nbuf, CHUNK, f), jnp.float32),
          pltpu.SemaphoreType.DMA((nbuf,)),
          pltpu.SemaphoreType.DMA((nbuf,)),
          pltpu.SemaphoreType.DMA((nbuf,)),
          pltpu.SemaphoreType.DMA((nbuf,)),
      ] + ([pltpu.VMEM_SHARED((n, f), jnp.float32)] if stage_y else []),
      compiler_params=pltpu.CompilerParams(use_tc_tiling_on_sc=tc_tiling),
  )


_CONTRACT_T = (((1,), (1,)), ((), ()))


def _mm_scale_body(x_ref, w_ref, b_ref, degp_ref, dinv_ref, y_ref):
  deg = (jnp.sum(degp_ref[0] + degp_ref[1], axis=-1, keepdims=True) * (1.0 / DW)
         + 1.0)
  dinv = lax.rsqrt(deg)
  dinv_ref[...] = dinv
  xw = lax.dot_general(x_ref[...], w_ref[...], _CONTRACT_T,
                       preferred_element_type=jnp.float32)
  y_ref[...] = dinv * (xw + b_ref[...])


def _layer2_body(acc_ref, y1_ref, dinv_ref, w_ref, b_ref, y2_ref):
  dinv = dinv_ref[...]
  h = jnp.maximum(dinv * (acc_ref[0] + acc_ref[1] + y1_ref[...]), 0.0)
  hw = lax.dot_general(h, w_ref[...], _CONTRACT_T,
                       preferred_element_type=jnp.float32)
  y2_ref[...] = dinv * (hw + b_ref[...])


def _final_body(acc_ref, y2_ref, dinv_ref, o_ref):
  z = dinv_ref[...] * (acc_ref[0] + acc_ref[1] + y2_ref[...])
  m = jnp.max(z, axis=1, keepdims=True)
  ez = jnp.exp(z - m)
  lse = jnp.log(jnp.sum(ez, axis=1, keepdims=True)) + m
  o_ref[...] = z - lse


def kernel(x, edge_index, W1, b1, W2, b2):
  n, f_in = x.shape
  e = edge_index.shape[1]
  f_hid = W1.shape[0]
  f_out = W2.shape[0]
  b1r = b1.reshape(1, f_hid)
  b2r = b2.reshape(1, f_out)

  degp = _make_deg(n, e)(edge_index)

  dinv, y1 = pl.pallas_call(
      _mm_scale_body,
      out_shape=[jax.ShapeDtypeStruct((n, 1), jnp.float32),
                 jax.ShapeDtypeStruct((n, f_hid), jnp.float32)],
  )(x, W1, b1r, degp)

  acc1 = _make_agg(n, e, f_hid, 3, tc_tiling=True)(y1, edge_index)

  y2 = pl.pallas_call(
      _layer2_body,
      out_shape=jax.ShapeDtypeStruct((n, f_out), jnp.float32),
  )(acc1, y1, dinv, W2, b2r)

  acc2 = _make_agg(n, e, f_out, 6, tc_tiling=False)(y2, edge_index)

  out = pl.pallas_call(
      _final_body,
      out_shape=jax.ShapeDtypeStruct((n, f_out), jnp.float32),
  )(acc2, y2, dinv)

  return out
